# baseline (speedup 1.0000x reference)
import jax
import jax.numpy as jnp
from jax.experimental import pallas as pl
from jax.experimental.pallas import tpu as pltpu

B, S = 2, 2048
D_MODEL = 1024
N_FEAT = 32
PROTO_DIM = 256
D_FEMB = 128
D_RH = 256
E = 8
DH = 1024
DH2 = 512
TC = 512
D_XIN = D_MODEL + D_FEMB


def _moe_kernel(
    hidden_ref, feat_ref, proto_ref,
    whctx_ref, wfctx_ref, wfeat_ref, bfeat_ref,
    wr1_ref, br1_ref, wr2_ref, br2_ref,
    we1_ref, be1_ref, we2_ref, be2_ref,
    delta_ref, gw_ref, gl_ref,
    xin_ref, hbuf_ref, fbuf_ref, sem_h, sem_f,
):
    e_idx = pl.program_id(0)
    h_idx = pl.program_id(1)
    b_idx = pl.program_id(2)
    first = jnp.logical_and(e_idx == 0, h_idx == 0)

    w1 = we1_ref[0].astype(jnp.bfloat16)
    w2 = we2_ref[0].astype(jnp.bfloat16)
    b1 = be1_ref[0]
    b2 = be2_ref[0]
    bias_on = jnp.where(h_idx == 0, 1.0, 0.0)

    @pl.when(first)
    def _router():
        proto = proto_ref[pl.ds(b_idx, 1), :]
        ctx_h = jnp.dot(proto, whctx_ref[...],
                        preferred_element_type=jnp.float32)
        ctx_f = jnp.dot(proto, wfctx_ref[...],
                        preferred_element_type=jnp.float32)
        for c in range(S // TC):
            off = c * TC
            cp_h = pltpu.make_async_copy(
                hidden_ref.at[b_idx, pl.ds(off, TC), :], hbuf_ref, sem_h)
            cp_f = pltpu.make_async_copy(
                feat_ref.at[b_idx, pl.ds(off, TC), :], fbuf_ref, sem_f)
            cp_h.start()
            cp_f.start()
            cp_h.wait()
            cp_f.wait()
            h = hbuf_ref[...]
            hidden_cond = h + ctx_h
            f = fbuf_ref[...]
            feat_cond = f + ctx_f
            femb = jnp.maximum(
                jnp.dot(feat_cond, wfeat_ref[...],
                        preferred_element_type=jnp.float32)
                + bfeat_ref[...], 0.0)
            xoff = b_idx * S + off
            xin_ref[pl.ds(xoff, TC), :D_MODEL] = hidden_cond.astype(jnp.bfloat16)
            xin_ref[pl.ds(xoff, TC), D_MODEL:] = femb.astype(jnp.bfloat16)

            router_in = jnp.concatenate([hidden_cond, feat_cond], axis=-1)
            r_h = jnp.maximum(
                jnp.dot(router_in, wr1_ref[...],
                        preferred_element_type=jnp.float32)
                + br1_ref[...], 0.0)
            logits = jnp.dot(r_h, wr2_ref[...],
                             preferred_element_type=jnp.float32) + br2_ref[...]
            gl_ref[b_idx, pl.ds(off, TC), :] = logits

            iota = jax.lax.broadcasted_iota(jnp.int32, logits.shape, 1)
            m1 = jnp.max(logits, axis=-1, keepdims=True)
            idx1 = jnp.min(jnp.where(logits == m1, iota, E),
                           axis=-1, keepdims=True)
            sel1 = iota == idx1
            logits2 = jnp.where(sel1, -jnp.inf, logits)
            m2 = jnp.max(logits2, axis=-1, keepdims=True)
            idx2 = jnp.min(jnp.where(logits2 == m2, iota, E),
                           axis=-1, keepdims=True)
            sel2 = iota == idx2
            e2v = jnp.exp(m2 - m1)
            wa = 1.0 / (1.0 + e2v)
            wb = e2v * wa
            gate = jnp.where(sel1, wa, 0.0) + jnp.where(sel2, wb, 0.0)
            gw_ref[b_idx, pl.ds(off, TC), :] = gate

    for c in range(S // TC):
        off = c * TC
        gate_blk = gw_ref[b_idx, pl.ds(off, TC), :]
        emask = (jax.lax.broadcasted_iota(jnp.int32, (1, E), 1) == e_idx)
        w_e = jnp.sum(jnp.where(emask, gate_blk, 0.0),
                      axis=-1, keepdims=True)

        xb = xin_ref[pl.ds(b_idx * S + off, TC), :]
        h1 = jnp.dot(xb, w1, preferred_element_type=jnp.float32)
        h1 = jnp.maximum(h1 + b1, 0.0).astype(jnp.bfloat16)
        oe = jnp.dot(h1, w2, preferred_element_type=jnp.float32)
        contrib = w_e * (oe + bias_on * b2)

        @pl.when(first)
        def _init():
            delta_ref[b_idx, pl.ds(off, TC), :] = contrib

        @pl.when(jnp.logical_not(first))
        def _acc():
            delta_ref[b_idx, pl.ds(off, TC), :] += contrib


@jax.jit
def kernel(hidden, feat, proto_context, W_hctx, W_fctx, W_feat, b_feat,
           W_r1, b_r1, W_r2, b_r2, W_e1, b_e1, W_e2, b_e2):
    grid = (E, DH // DH2, B)

    def full3(e, h, b):
        return (0, 0, 0)

    def rep2(e, h, b):
        return (0, 0)

    def w1map(e, h, b):
        return (e, 0, h)

    def b1map(e, h, b):
        return (e, 0, h)

    def w2map(e, h, b):
        return (e, h, 0)

    def b2map(e, h, b):
        return (e, 0, 0)

    in_specs = [
        pl.BlockSpec(memory_space=pl.ANY),
        pl.BlockSpec(memory_space=pl.ANY),
        pl.BlockSpec((B, PROTO_DIM), rep2),
        pl.BlockSpec((PROTO_DIM, D_MODEL), rep2),
        pl.BlockSpec((PROTO_DIM, N_FEAT), rep2),
        pl.BlockSpec((N_FEAT, D_FEMB), rep2),
        pl.BlockSpec((1, D_FEMB), rep2),
        pl.BlockSpec((D_MODEL + N_FEAT, D_RH), rep2),
        pl.BlockSpec((1, D_RH), rep2),
        pl.BlockSpec((D_RH, E), rep2),
        pl.BlockSpec((1, E), rep2),
        pl.BlockSpec((1, D_XIN, DH2), w1map),
        pl.BlockSpec((1, 1, DH2), b1map),
        pl.BlockSpec((1, DH2, D_MODEL), w2map),
        pl.BlockSpec((1, 1, D_MODEL), b2map),
    ]
    out_specs = [
        pl.BlockSpec((B, S, D_MODEL), full3),
        pl.BlockSpec((B, S, E), full3),
        pl.BlockSpec((B, S, E), full3),
    ]
    out_shape = [
        jax.ShapeDtypeStruct((B, S, D_MODEL), jnp.float32),
        jax.ShapeDtypeStruct((B, S, E), jnp.float32),
        jax.ShapeDtypeStruct((B, S, E), jnp.float32),
    ]

    delta, gate_weights, gate_logits = pl.pallas_call(
        _moe_kernel,
        grid=grid,
        in_specs=in_specs,
        out_specs=out_specs,
        out_shape=out_shape,
        scratch_shapes=[
            pltpu.VMEM((B * S, D_XIN), jnp.bfloat16),
            pltpu.VMEM((TC, D_MODEL), jnp.float32),
            pltpu.VMEM((TC, N_FEAT), jnp.float32),
            pltpu.SemaphoreType.DMA,
            pltpu.SemaphoreType.DMA,
        ],
    )(
        hidden, feat, proto_context,
        W_hctx, W_fctx, W_feat, b_feat.reshape(1, D_FEMB),
        W_r1, b_r1.reshape(1, D_RH), W_r2, b_r2.reshape(1, E),
        W_e1, b_e1.reshape(E, 1, DH), W_e2, b_e2.reshape(E, 1, D_MODEL),
    )
    return delta, gate_weights, gate_logits


# speedup vs baseline: 1.0725x; 1.0725x over previous
import jax
import jax.numpy as jnp
from jax.experimental import pallas as pl
from jax.experimental.pallas import tpu as pltpu

B, S = 2, 2048
D_MODEL = 1024
N_FEAT = 32
PROTO_DIM = 256
D_FEMB = 128
D_RH = 256
E = 8
DH = 1024
DH2 = 512
TC = 1024
D_XIN = D_MODEL + D_FEMB


def _moe_kernel(
    hidden_ref, feat_ref, proto_ref,
    whctx_ref, wfctx_ref, wfeat_ref, bfeat_ref,
    wr1_ref, br1_ref, wr2_ref, br2_ref,
    we1_ref, be1_ref, we2_ref, be2_ref,
    delta_ref, gw_ref, gl_ref,
    xin_ref, hbuf_ref, fbuf_ref, sem_h, sem_f,
):
    e_idx = pl.program_id(0)
    h_idx = pl.program_id(1)
    b_idx = pl.program_id(2)
    first = jnp.logical_and(e_idx == 0, h_idx == 0)

    w1 = we1_ref[0].astype(jnp.bfloat16)
    w2 = we2_ref[0].astype(jnp.bfloat16)
    b1 = be1_ref[0]
    b2 = be2_ref[0]
    bias_on = jnp.where(h_idx == 0, 1.0, 0.0)

    @pl.when(first)
    def _router():
        proto = proto_ref[pl.ds(b_idx, 1), :]
        ctx_h = jnp.dot(proto, whctx_ref[...],
                        preferred_element_type=jnp.float32)
        ctx_f = jnp.dot(proto, wfctx_ref[...],
                        preferred_element_type=jnp.float32)
        for c in range(S // TC):
            off = c * TC
            cp_h = pltpu.make_async_copy(
                hidden_ref.at[b_idx, pl.ds(off, TC), :], hbuf_ref, sem_h)
            cp_f = pltpu.make_async_copy(
                feat_ref.at[b_idx, pl.ds(off, TC), :], fbuf_ref, sem_f)
            cp_h.start()
            cp_f.start()
            cp_h.wait()
            cp_f.wait()
            h = hbuf_ref[...]
            hidden_cond = h + ctx_h
            f = fbuf_ref[...]
            feat_cond = f + ctx_f
            femb = jnp.maximum(
                jnp.dot(feat_cond, wfeat_ref[...],
                        preferred_element_type=jnp.float32)
                + bfeat_ref[...], 0.0)
            xoff = b_idx * S + off
            xin_ref[pl.ds(xoff, TC), :D_MODEL] = hidden_cond.astype(jnp.bfloat16)
            xin_ref[pl.ds(xoff, TC), D_MODEL:] = femb.astype(jnp.bfloat16)

            router_in = jnp.concatenate([hidden_cond, feat_cond], axis=-1)
            r_h = jnp.maximum(
                jnp.dot(router_in, wr1_ref[...],
                        preferred_element_type=jnp.float32)
                + br1_ref[...], 0.0)
            logits = jnp.dot(r_h, wr2_ref[...],
                             preferred_element_type=jnp.float32) + br2_ref[...]
            gl_ref[b_idx, pl.ds(off, TC), :] = logits

            iota = jax.lax.broadcasted_iota(jnp.int32, logits.shape, 1)
            m1 = jnp.max(logits, axis=-1, keepdims=True)
            idx1 = jnp.min(jnp.where(logits == m1, iota, E),
                           axis=-1, keepdims=True)
            sel1 = iota == idx1
            logits2 = jnp.where(sel1, -jnp.inf, logits)
            m2 = jnp.max(logits2, axis=-1, keepdims=True)
            idx2 = jnp.min(jnp.where(logits2 == m2, iota, E),
                           axis=-1, keepdims=True)
            sel2 = iota == idx2
            e2v = jnp.exp(m2 - m1)
            wa = 1.0 / (1.0 + e2v)
            wb = e2v * wa
            gate = jnp.where(sel1, wa, 0.0) + jnp.where(sel2, wb, 0.0)
            gw_ref[b_idx, pl.ds(off, TC), :] = gate

    for c in range(S // TC):
        off = c * TC
        gate_blk = gw_ref[b_idx, pl.ds(off, TC), :]
        emask = (jax.lax.broadcasted_iota(jnp.int32, (1, E), 1) == e_idx)
        w_e = jnp.sum(jnp.where(emask, gate_blk, 0.0),
                      axis=-1, keepdims=True)

        xb = xin_ref[pl.ds(b_idx * S + off, TC), :]
        h1 = jnp.dot(xb, w1, preferred_element_type=jnp.float32)
        h1 = jnp.maximum(h1 + b1, 0.0).astype(jnp.bfloat16)
        oe = jnp.dot(h1, w2, preferred_element_type=jnp.float32)
        contrib = w_e * (oe + bias_on * b2)

        @pl.when(first)
        def _init():
            delta_ref[b_idx, pl.ds(off, TC), :] = contrib

        @pl.when(jnp.logical_not(first))
        def _acc():
            delta_ref[b_idx, pl.ds(off, TC), :] += contrib


@jax.jit
def kernel(hidden, feat, proto_context, W_hctx, W_fctx, W_feat, b_feat,
           W_r1, b_r1, W_r2, b_r2, W_e1, b_e1, W_e2, b_e2):
    grid = (E, DH // DH2, B)

    def full3(e, h, b):
        return (0, 0, 0)

    def rep2(e, h, b):
        return (0, 0)

    def w1map(e, h, b):
        return (e, 0, h)

    def b1map(e, h, b):
        return (e, 0, h)

    def w2map(e, h, b):
        return (e, h, 0)

    def b2map(e, h, b):
        return (e, 0, 0)

    in_specs = [
        pl.BlockSpec(memory_space=pl.ANY),
        pl.BlockSpec(memory_space=pl.ANY),
        pl.BlockSpec((B, PROTO_DIM), rep2),
        pl.BlockSpec((PROTO_DIM, D_MODEL), rep2),
        pl.BlockSpec((PROTO_DIM, N_FEAT), rep2),
        pl.BlockSpec((N_FEAT, D_FEMB), rep2),
        pl.BlockSpec((1, D_FEMB), rep2),
        pl.BlockSpec((D_MODEL + N_FEAT, D_RH), rep2),
        pl.BlockSpec((1, D_RH), rep2),
        pl.BlockSpec((D_RH, E), rep2),
        pl.BlockSpec((1, E), rep2),
        pl.BlockSpec((1, D_XIN, DH2), w1map),
        pl.BlockSpec((1, 1, DH2), b1map),
        pl.BlockSpec((1, DH2, D_MODEL), w2map),
        pl.BlockSpec((1, 1, D_MODEL), b2map),
    ]
    out_specs = [
        pl.BlockSpec((B, S, D_MODEL), full3),
        pl.BlockSpec((B, S, E), full3),
        pl.BlockSpec((B, S, E), full3),
    ]
    out_shape = [
        jax.ShapeDtypeStruct((B, S, D_MODEL), jnp.float32),
        jax.ShapeDtypeStruct((B, S, E), jnp.float32),
        jax.ShapeDtypeStruct((B, S, E), jnp.float32),
    ]

    delta, gate_weights, gate_logits = pl.pallas_call(
        _moe_kernel,
        grid=grid,
        in_specs=in_specs,
        out_specs=out_specs,
        out_shape=out_shape,
        scratch_shapes=[
            pltpu.VMEM((B * S, D_XIN), jnp.bfloat16),
            pltpu.VMEM((TC, D_MODEL), jnp.float32),
            pltpu.VMEM((TC, N_FEAT), jnp.float32),
            pltpu.SemaphoreType.DMA,
            pltpu.SemaphoreType.DMA,
        ],
    )(
        hidden, feat, proto_context,
        W_hctx, W_fctx, W_feat, b_feat.reshape(1, D_FEMB),
        W_r1, b_r1.reshape(1, D_RH), W_r2, b_r2.reshape(1, E),
        W_e1, b_e1.reshape(E, 1, DH), W_e2, b_e2.reshape(E, 1, D_MODEL),
    )
    return delta, gate_weights, gate_logits


# speedup vs baseline: 1.0744x; 1.0017x over previous
import jax
import jax.numpy as jnp
from jax.experimental import pallas as pl
from jax.experimental.pallas import tpu as pltpu

B, S = 2, 2048
D_MODEL = 1024
N_FEAT = 32
PROTO_DIM = 256
D_FEMB = 128
D_RH = 256
E = 8
DH = 1024
DH2 = 512
TC = 2048
RC = 512
D_XIN = D_MODEL + D_FEMB


def _moe_kernel(
    hidden_ref, feat_ref, proto_ref,
    whctx_ref, wfctx_ref, wfeat_ref, bfeat_ref,
    wr1_ref, br1_ref, wr2_ref, br2_ref,
    we1_ref, be1_ref, we2_ref, be2_ref,
    delta_ref, gw_ref, gl_ref,
    xin_ref, hbuf_ref, fbuf_ref, sem_h, sem_f,
):
    e_idx = pl.program_id(0)
    h_idx = pl.program_id(1)
    b_idx = pl.program_id(2)
    first = jnp.logical_and(e_idx == 0, h_idx == 0)

    w1 = we1_ref[0].astype(jnp.bfloat16)
    w2 = we2_ref[0].astype(jnp.bfloat16)
    b1 = be1_ref[0]
    b2 = be2_ref[0]
    bias_on = jnp.where(h_idx == 0, 1.0, 0.0)

    @pl.when(first)
    def _router():
        proto = proto_ref[pl.ds(b_idx, 1), :]
        ctx_h = jnp.dot(proto, whctx_ref[...],
                        preferred_element_type=jnp.float32)
        ctx_f = jnp.dot(proto, wfctx_ref[...],
                        preferred_element_type=jnp.float32)
        for c in range(S // RC):
            off = c * RC
            cp_h = pltpu.make_async_copy(
                hidden_ref.at[b_idx, pl.ds(off, RC), :], hbuf_ref, sem_h)
            cp_f = pltpu.make_async_copy(
                feat_ref.at[b_idx, pl.ds(off, RC), :], fbuf_ref, sem_f)
            cp_h.start()
            cp_f.start()
            cp_h.wait()
            cp_f.wait()
            h = hbuf_ref[...]
            hidden_cond = h + ctx_h
            f = fbuf_ref[...]
            feat_cond = f + ctx_f
            femb = jnp.maximum(
                jnp.dot(feat_cond, wfeat_ref[...],
                        preferred_element_type=jnp.float32)
                + bfeat_ref[...], 0.0)
            xoff = b_idx * S + off
            xin_ref[pl.ds(xoff, RC), :D_MODEL] = hidden_cond.astype(jnp.bfloat16)
            xin_ref[pl.ds(xoff, RC), D_MODEL:] = femb.astype(jnp.bfloat16)

            router_in = jnp.concatenate([hidden_cond, feat_cond], axis=-1)
            r_h = jnp.maximum(
                jnp.dot(router_in, wr1_ref[...],
                        preferred_element_type=jnp.float32)
                + br1_ref[...], 0.0)
            logits = jnp.dot(r_h, wr2_ref[...],
                             preferred_element_type=jnp.float32) + br2_ref[...]
            gl_ref[b_idx, pl.ds(off, RC), :] = logits

            iota = jax.lax.broadcasted_iota(jnp.int32, logits.shape, 1)
            m1 = jnp.max(logits, axis=-1, keepdims=True)
            idx1 = jnp.min(jnp.where(logits == m1, iota, E),
                           axis=-1, keepdims=True)
            sel1 = iota == idx1
            logits2 = jnp.where(sel1, -jnp.inf, logits)
            m2 = jnp.max(logits2, axis=-1, keepdims=True)
            idx2 = jnp.min(jnp.where(logits2 == m2, iota, E),
                           axis=-1, keepdims=True)
            sel2 = iota == idx2
            e2v = jnp.exp(m2 - m1)
            wa = 1.0 / (1.0 + e2v)
            wb = e2v * wa
            gate = jnp.where(sel1, wa, 0.0) + jnp.where(sel2, wb, 0.0)
            gw_ref[b_idx, pl.ds(off, RC), :] = gate

    for c in range(S // TC):
        off = c * TC
        gate_blk = gw_ref[b_idx, pl.ds(off, TC), :]
        emask = (jax.lax.broadcasted_iota(jnp.int32, (1, E), 1) == e_idx)
        w_e = jnp.sum(jnp.where(emask, gate_blk, 0.0),
                      axis=-1, keepdims=True)

        xb = xin_ref[pl.ds(b_idx * S + off, TC), :]
        h1 = jnp.dot(xb, w1, preferred_element_type=jnp.float32)
        h1 = jnp.maximum(h1 + b1, 0.0).astype(jnp.bfloat16)
        oe = jnp.dot(h1, w2, preferred_element_type=jnp.float32)
        contrib = w_e * (oe + bias_on * b2)

        @pl.when(first)
        def _init():
            delta_ref[b_idx, pl.ds(off, TC), :] = contrib

        @pl.when(jnp.logical_not(first))
        def _acc():
            delta_ref[b_idx, pl.ds(off, TC), :] += contrib


@jax.jit
def kernel(hidden, feat, proto_context, W_hctx, W_fctx, W_feat, b_feat,
           W_r1, b_r1, W_r2, b_r2, W_e1, b_e1, W_e2, b_e2):
    grid = (E, DH // DH2, B)

    def full3(e, h, b):
        return (0, 0, 0)

    def rep2(e, h, b):
        return (0, 0)

    def w1map(e, h, b):
        return (e, 0, h)

    def b1map(e, h, b):
        return (e, 0, h)

    def w2map(e, h, b):
        return (e, h, 0)

    def b2map(e, h, b):
        return (e, 0, 0)

    in_specs = [
        pl.BlockSpec(memory_space=pl.ANY),
        pl.BlockSpec(memory_space=pl.ANY),
        pl.BlockSpec((B, PROTO_DIM), rep2),
        pl.BlockSpec((PROTO_DIM, D_MODEL), rep2),
        pl.BlockSpec((PROTO_DIM, N_FEAT), rep2),
        pl.BlockSpec((N_FEAT, D_FEMB), rep2),
        pl.BlockSpec((1, D_FEMB), rep2),
        pl.BlockSpec((D_MODEL + N_FEAT, D_RH), rep2),
        pl.BlockSpec((1, D_RH), rep2),
        pl.BlockSpec((D_RH, E), rep2),
        pl.BlockSpec((1, E), rep2),
        pl.BlockSpec((1, D_XIN, DH2), w1map),
        pl.BlockSpec((1, 1, DH2), b1map),
        pl.BlockSpec((1, DH2, D_MODEL), w2map),
        pl.BlockSpec((1, 1, D_MODEL), b2map),
    ]
    out_specs = [
        pl.BlockSpec((B, S, D_MODEL), full3),
        pl.BlockSpec((B, S, E), full3),
        pl.BlockSpec((B, S, E), full3),
    ]
    out_shape = [
        jax.ShapeDtypeStruct((B, S, D_MODEL), jnp.float32),
        jax.ShapeDtypeStruct((B, S, E), jnp.float32),
        jax.ShapeDtypeStruct((B, S, E), jnp.float32),
    ]

    delta, gate_weights, gate_logits = pl.pallas_call(
        _moe_kernel,
        grid=grid,
        in_specs=in_specs,
        out_specs=out_specs,
        out_shape=out_shape,
        scratch_shapes=[
            pltpu.VMEM((B * S, D_XIN), jnp.bfloat16),
            pltpu.VMEM((RC, D_MODEL), jnp.float32),
            pltpu.VMEM((RC, N_FEAT), jnp.float32),
            pltpu.SemaphoreType.DMA,
            pltpu.SemaphoreType.DMA,
        ],
    )(
        hidden, feat, proto_context,
        W_hctx, W_fctx, W_feat, b_feat.reshape(1, D_FEMB),
        W_r1, b_r1.reshape(1, D_RH), W_r2, b_r2.reshape(1, E),
        W_e1, b_e1.reshape(E, 1, DH), W_e2, b_e2.reshape(E, 1, D_MODEL),
    )
    return delta, gate_weights, gate_logits
